# baseline (speedup 1.0000x reference)
import functools

import jax
import jax.numpy as jnp
from jax import lax
from jax.experimental import pallas as pl
from jax.experimental.pallas import tpu as pltpu
from jax.experimental.pallas import tpu_sc as plsc

N_NODES = 10000
N_EDGES = 320000
D = 128

_NC = 2
_NS = 16
_NW = _NC * _NS
_EPT = N_EDGES // _NW
_WIN = 80
_NWIN = _EPT // _WIN
_NPAD = 10240
_RPT = _NPAD // _NS
_SHIFT = 14


@functools.partial(
    pl.kernel,
    out_type=(
        jax.ShapeDtypeStruct((_NC, _NPAD, D), jnp.float32),
        jax.ShapeDtypeStruct((_NW, _NPAD), jnp.float32),
    ),
    mesh=plsc.VectorSubcoreMesh(core_axis_name="c", subcore_axis_name="s"),
    compiler_params=pltpu.CompilerParams(needs_layout_passes=False),
    scratch_types=[
        pltpu.VMEM((_EPT,), jnp.int32),
        pltpu.VMEM((_WIN,), jnp.int32),
        pltpu.VMEM((_WIN,), jnp.int32),
        pltpu.VMEM((_WIN,), jnp.int32),
        pltpu.VMEM((_WIN,), jnp.int32),
        pltpu.VMEM((_WIN, D), jnp.float32),
        pltpu.VMEM((_WIN, D), jnp.float32),
        pltpu.VMEM((_NPAD,), jnp.float32),
        pltpu.VMEM_SHARED((_NPAD, D), jnp.float32),
        pltpu.SemaphoreType.DMA,
        pltpu.SemaphoreType.DMA,
    ],
)
def _sc_segment_sum(x_hbm, pk_hbm, zeros_hbm, psum_hbm, cnt_hbm,
                    pk_t, src_v0, src_v1, dst_v0, dst_v1, rows_v0, rows_v1,
                    cnt_v, acc, semg, sems):
    c = lax.axis_index("c")
    s = lax.axis_index("s")
    wid = c * _NS + s
    base0 = wid * _EPT
    src_v = (src_v0, src_v1)
    dst_v = (dst_v0, dst_v1)
    rows_v = (rows_v0, rows_v1)

    pltpu.async_copy(pk_hbm.at[pl.ds(base0, _EPT)], pk_t, semg)
    pltpu.sync_copy(zeros_hbm, acc.at[pl.ds(s * _RPT, _RPT)])

    zeros16 = jnp.zeros((16,), jnp.float32)

    def zbody(i, carry):
        cnt_v[pl.ds(i * 16, 16)] = zeros16
        return carry

    lax.fori_loop(0, _NPAD // 16, zbody, 0)
    pltpu.make_async_copy(pk_hbm.at[pl.ds(base0, _EPT)], pk_t, semg).wait()
    plsc.subcore_barrier()

    ones16 = jnp.ones((16,), jnp.float32)
    mask14 = jnp.full((16,), (1 << _SHIFT) - 1, jnp.int32)

    def extract(w, p):
        for j in range(_WIN // 16):
            pk = pk_t[pl.ds(w * _WIN + j * 16, 16)]
            src_v[p][pl.ds(j * 16, 16)] = lax.bitwise_and(pk, mask14)
            dst_v[p][pl.ds(j * 16, 16)] = lax.shift_right_logical(pk, _SHIFT)

    def launch_gather(p):
        pltpu.async_copy(x_hbm.at[src_v[p]], rows_v[p], semg)

    def wait_gather(p):
        pltpu.make_async_copy(x_hbm.at[src_v[p]], rows_v[p], semg).wait()

    def launch_scatter(p):
        pltpu.async_copy(rows_v[p], acc.at[dst_v[p]], sems, add=True)

    def wait_scatter(p):
        pltpu.make_async_copy(rows_v[p], acc.at[dst_v[p]], sems).wait()

    def hist(p):
        for j in range(_WIN // 16):
            dvec = dst_v[p][pl.ds(j * 16, 16)]
            plsc.addupdate_scatter(cnt_v, [dvec], ones16)

    def sub_iter(w, p, first=False):
        wait_gather(p)
        if not first:
            wait_scatter(1 - p)

        @pl.when(w + 1 < _NWIN)
        def _():
            extract(w + 1, 1 - p)
            launch_gather(1 - p)

        hist(p)
        launch_scatter(p)

    extract(0, 0)
    launch_gather(0)
    sub_iter(0, 0, first=True)

    def pair_odd(i, carry):
        sub_iter(2 * i + 1, 1)
        sub_iter(2 * i + 2, 0)
        return carry

    lax.fori_loop(0, (_NWIN - 1) // 2, pair_odd, 0)
    wait_scatter(0)

    plsc.subcore_barrier()
    pltpu.sync_copy(acc.at[pl.ds(s * _RPT, _RPT)],
                    psum_hbm.at[c, pl.ds(s * _RPT, _RPT)])
    pltpu.sync_copy(cnt_v, cnt_hbm.at[wid])


def _tc_body(psum_ref, cnts_ref, x_ref, wlt_ref, wrt_ref, bl_ref, g_ref,
             b_ref, o_ref):
    ones = jnp.ones((_NW, 1), jnp.float32)
    cnt = lax.dot_general(cnts_ref[...], ones, (((0,), (0,)), ((), ())),
                          preferred_element_type=jnp.float32)
    p = psum_ref[0] + psum_ref[1]
    agg = p / jnp.maximum(cnt, 1.0)
    h = (jnp.dot(agg, wlt_ref[...], preferred_element_type=jnp.float32)
         + jnp.dot(x_ref[...], wrt_ref[...], preferred_element_type=jnp.float32)
         + bl_ref[...])
    mean = jnp.mean(h, axis=1, keepdims=True)
    d = h - mean
    var = jnp.mean(d * d, axis=1, keepdims=True)
    hn = d * lax.rsqrt(var + 1e-5) * g_ref[...] + b_ref[...]
    o_ref[...] = 0.5 * hn * (1.0 + lax.erf(hn * 0.7071067811865476))


def _tc_epilogue(psum, cnts, x, wlt, wrt, bl, g, b):
    bn = 1024
    grid = ((N_NODES + bn - 1) // bn,)
    return pl.pallas_call(
        _tc_body,
        grid=grid,
        in_specs=[
            pl.BlockSpec((_NC, bn, D), lambda i: (0, i, 0)),
            pl.BlockSpec((_NW, bn), lambda i: (0, i)),
            pl.BlockSpec((bn, D), lambda i: (i, 0)),
            pl.BlockSpec((D, D), lambda i: (0, 0)),
            pl.BlockSpec((D, D), lambda i: (0, 0)),
            pl.BlockSpec((1, D), lambda i: (0, 0)),
            pl.BlockSpec((1, D), lambda i: (0, 0)),
            pl.BlockSpec((1, D), lambda i: (0, 0)),
        ],
        out_specs=pl.BlockSpec((bn, D), lambda i: (i, 0)),
        out_shape=jax.ShapeDtypeStruct((N_NODES, D), jnp.float32),
    )(psum, cnts, x, wlt, wrt, bl, g, b)


def kernel(x, edge_index, W_l, b_l, W_r, ln_gamma, ln_beta):
    ei = edge_index.astype(jnp.int32)
    packed = ei[0] | (ei[1] << _SHIFT)
    zeros = jnp.zeros((_RPT, D), jnp.float32)
    psum, cnts = _sc_segment_sum(x, packed, zeros)
    return _tc_epilogue(psum, cnts, x, W_l.T, W_r.T,
                        b_l.reshape(1, D), ln_gamma.reshape(1, D),
                        ln_beta.reshape(1, D))


# speedup vs baseline: 1.0474x; 1.0474x over previous
import functools

import jax
import jax.numpy as jnp
from jax import lax
from jax.experimental import pallas as pl
from jax.experimental.pallas import tpu as pltpu
from jax.experimental.pallas import tpu_sc as plsc

N_NODES = 10000
N_EDGES = 320000
D = 128

_NC = 2
_NS = 16
_NW = _NC * _NS
_EPT = N_EDGES // _NW
_WIN = 80
_NWIN = _EPT // _WIN
_NPAD = 10240
_RPT = _NPAD // _NS
_SHIFT = 14


@functools.partial(
    pl.kernel,
    out_type=(
        jax.ShapeDtypeStruct((_NC, _NPAD, D), jnp.bfloat16),
        jax.ShapeDtypeStruct((_NW, _NPAD), jnp.float32),
    ),
    mesh=plsc.VectorSubcoreMesh(core_axis_name="c", subcore_axis_name="s"),
    compiler_params=pltpu.CompilerParams(needs_layout_passes=False,
                                         use_tc_tiling_on_sc=False),
    scratch_types=[
        pltpu.VMEM((_EPT,), jnp.int32),
        pltpu.VMEM((_WIN,), jnp.int32),
        pltpu.VMEM((_WIN,), jnp.int32),
        pltpu.VMEM((_WIN,), jnp.int32),
        pltpu.VMEM((_WIN,), jnp.int32),
        pltpu.VMEM((_WIN, D), jnp.bfloat16),
        pltpu.VMEM((_WIN, D), jnp.bfloat16),
        pltpu.VMEM((_NPAD,), jnp.float32),
        pltpu.VMEM_SHARED((_NPAD, D), jnp.bfloat16),
        pltpu.SemaphoreType.DMA,
        pltpu.SemaphoreType.DMA,
    ],
)
def _sc_segment_sum(x_hbm, pk_hbm, zeros_hbm, psum_hbm, cnt_hbm,
                    pk_t, src_v0, src_v1, dst_v0, dst_v1, rows_v0, rows_v1,
                    cnt_v, acc, semg, sems):
    c = lax.axis_index("c")
    s = lax.axis_index("s")
    wid = c * _NS + s
    base0 = wid * _EPT
    src_v = (src_v0, src_v1)
    dst_v = (dst_v0, dst_v1)
    rows_v = (rows_v0, rows_v1)

    pltpu.async_copy(pk_hbm.at[pl.ds(base0, _EPT)], pk_t, semg)
    pltpu.sync_copy(zeros_hbm, acc.at[pl.ds(s * _RPT, _RPT)])

    zeros16 = jnp.zeros((16,), jnp.float32)

    def zbody(i, carry):
        cnt_v[pl.ds(i * 16, 16)] = zeros16
        return carry

    lax.fori_loop(0, _NPAD // 16, zbody, 0)
    pltpu.make_async_copy(pk_hbm.at[pl.ds(base0, _EPT)], pk_t, semg).wait()
    plsc.subcore_barrier()

    ones16 = jnp.ones((16,), jnp.float32)
    mask14 = jnp.full((16,), (1 << _SHIFT) - 1, jnp.int32)

    def extract(w, p):
        for j in range(_WIN // 16):
            pk = pk_t[pl.ds(w * _WIN + j * 16, 16)]
            src_v[p][pl.ds(j * 16, 16)] = lax.bitwise_and(pk, mask14)
            dst_v[p][pl.ds(j * 16, 16)] = lax.shift_right_logical(pk, _SHIFT)

    def launch_gather(p):
        pltpu.async_copy(x_hbm.at[src_v[p]], rows_v[p], semg)

    def wait_gather(p):
        pltpu.make_async_copy(x_hbm.at[src_v[p]], rows_v[p], semg).wait()

    def launch_scatter(p):
        pltpu.async_copy(rows_v[p], acc.at[dst_v[p]], sems, add=True)

    def wait_scatter(p):
        pltpu.make_async_copy(rows_v[p], acc.at[dst_v[p]], sems).wait()

    def hist(p):
        for j in range(_WIN // 16):
            dvec = dst_v[p][pl.ds(j * 16, 16)]
            plsc.addupdate_scatter(cnt_v, [dvec], ones16)

    def sub_iter(w, p, first=False):
        wait_gather(p)
        if not first:
            wait_scatter(1 - p)

        @pl.when(w + 1 < _NWIN)
        def _():
            extract(w + 1, 1 - p)
            launch_gather(1 - p)

        hist(p)
        launch_scatter(p)

    extract(0, 0)
    launch_gather(0)
    sub_iter(0, 0, first=True)

    def pair_odd(i, carry):
        sub_iter(2 * i + 1, 1)
        sub_iter(2 * i + 2, 0)
        return carry

    lax.fori_loop(0, (_NWIN - 1) // 2, pair_odd, 0)
    wait_scatter(0)

    plsc.subcore_barrier()
    pltpu.sync_copy(acc.at[pl.ds(s * _RPT, _RPT)],
                    psum_hbm.at[c, pl.ds(s * _RPT, _RPT)])
    pltpu.sync_copy(cnt_v, cnt_hbm.at[wid])


def _tc_body(psum_ref, cnts_ref, x_ref, wlt_ref, wrt_ref, bl_ref, g_ref,
             b_ref, o_ref):
    ones = jnp.ones((_NW, 1), jnp.float32)
    cnt = lax.dot_general(cnts_ref[...], ones, (((0,), (0,)), ((), ())),
                          preferred_element_type=jnp.float32)
    p = (psum_ref[0].astype(jnp.float32)
         + psum_ref[1].astype(jnp.float32))
    agg = p / jnp.maximum(cnt, 1.0)
    h = (jnp.dot(agg, wlt_ref[...], preferred_element_type=jnp.float32)
         + jnp.dot(x_ref[...], wrt_ref[...], preferred_element_type=jnp.float32)
         + bl_ref[...])
    mean = jnp.mean(h, axis=1, keepdims=True)
    d = h - mean
    var = jnp.mean(d * d, axis=1, keepdims=True)
    hn = d * lax.rsqrt(var + 1e-5) * g_ref[...] + b_ref[...]
    o_ref[...] = 0.5 * hn * (1.0 + lax.erf(hn * 0.7071067811865476))


def _tc_epilogue(psum, cnts, x, wlt, wrt, bl, g, b):
    bn = 1024
    grid = ((N_NODES + bn - 1) // bn,)
    return pl.pallas_call(
        _tc_body,
        grid=grid,
        in_specs=[
            pl.BlockSpec((_NC, bn, D), lambda i: (0, i, 0)),
            pl.BlockSpec((_NW, bn), lambda i: (0, i)),
            pl.BlockSpec((bn, D), lambda i: (i, 0)),
            pl.BlockSpec((D, D), lambda i: (0, 0)),
            pl.BlockSpec((D, D), lambda i: (0, 0)),
            pl.BlockSpec((1, D), lambda i: (0, 0)),
            pl.BlockSpec((1, D), lambda i: (0, 0)),
            pl.BlockSpec((1, D), lambda i: (0, 0)),
        ],
        out_specs=pl.BlockSpec((bn, D), lambda i: (i, 0)),
        out_shape=jax.ShapeDtypeStruct((N_NODES, D), jnp.float32),
    )(psum, cnts, x, wlt, wrt, bl, g, b)


def kernel(x, edge_index, W_l, b_l, W_r, ln_gamma, ln_beta):
    ei = edge_index.astype(jnp.int32)
    packed = ei[0] | (ei[1] << _SHIFT)
    zeros = jnp.zeros((_RPT, D), jnp.bfloat16)
    psum, cnts = _sc_segment_sum(x.astype(jnp.bfloat16), packed, zeros)
    return _tc_epilogue(psum, cnts, x, W_l.T, W_r.T,
                        b_l.reshape(1, D), ln_gamma.reshape(1, D),
                        ln_beta.reshape(1, D))


# speedup vs baseline: 1.4042x; 1.3406x over previous
import functools

import jax
import jax.numpy as jnp
from jax import lax
from jax.experimental import pallas as pl
from jax.experimental.pallas import tpu as pltpu
from jax.experimental.pallas import tpu_sc as plsc

N_NODES = 10000
N_EDGES = 320000
D = 128

_NC = 2
_NS = 16
_NW = _NC * _NS
_WIN = 128
_NWIN = 80
_EPT = _WIN * _NWIN
_EPAD = _NW * _EPT
_NPAD = 10240
_RPT = _NPAD // _NS
_SHIFT = 14


@functools.partial(
    pl.kernel,
    out_type=(
        jax.ShapeDtypeStruct((_NC, _NPAD, D), jnp.bfloat16),
        jax.ShapeDtypeStruct((_NW, _NPAD), jnp.float32),
    ),
    mesh=plsc.VectorSubcoreMesh(core_axis_name="c", subcore_axis_name="s"),
    compiler_params=pltpu.CompilerParams(needs_layout_passes=False,
                                         use_tc_tiling_on_sc=False),
    scratch_types=[
        pltpu.VMEM((_EPT,), jnp.int32),
    ] + [pltpu.VMEM((_WIN,), jnp.int32) for _ in range(4)]
      + [pltpu.VMEM((_WIN,), jnp.int32) for _ in range(4)]
      + [pltpu.VMEM((_WIN, D), jnp.bfloat16) for _ in range(4)]
      + [
        pltpu.VMEM((_NPAD,), jnp.float32),
        pltpu.VMEM_SHARED((_NPAD, D), jnp.bfloat16),
    ] + [pltpu.SemaphoreType.DMA for _ in range(4)]
      + [pltpu.SemaphoreType.DMA for _ in range(2)],
)
def _sc_segment_sum(x_hbm, pk_hbm, zeros_hbm, psum_hbm, cnt_hbm,
                    pk_t, s0, s1, s2, s3, d0, d1, d2, d3, r0, r1, r2, r3,
                    cnt_v, acc, g0, g1, g2, g3, t0, t1):
    c = lax.axis_index("c")
    s = lax.axis_index("s")
    wid = c * _NS + s
    base0 = wid * _EPT
    src_v = (s0, s1, s2, s3)
    dst_v = (d0, d1, d2, d3)
    rows_v = (r0, r1, r2, r3)
    semg = (g0, g1, g2, g3)
    sems = (t0, t1)

    pltpu.async_copy(pk_hbm.at[pl.ds(base0, _EPT)], pk_t, g0)
    pltpu.sync_copy(zeros_hbm, acc.at[pl.ds(s * _RPT, _RPT)])

    zeros16 = jnp.zeros((16,), jnp.float32)

    def zbody(i, carry):
        cnt_v[pl.ds(i * 16, 16)] = zeros16
        return carry

    lax.fori_loop(0, _NPAD // 16, zbody, 0)
    pltpu.make_async_copy(pk_hbm.at[pl.ds(base0, _EPT)], pk_t, g0).wait()
    plsc.subcore_barrier()

    ones16 = jnp.ones((16,), jnp.float32)
    mask14 = jnp.full((16,), (1 << _SHIFT) - 1, jnp.int32)

    def extract(w, p):
        for j in range(_WIN // 16):
            pk = pk_t[pl.ds(w * _WIN + j * 16, 16)]
            src_v[p][pl.ds(j * 16, 16)] = lax.bitwise_and(pk, mask14)
            dst_v[p][pl.ds(j * 16, 16)] = lax.shift_right_logical(pk, _SHIFT)

    def launch_gather(p):
        pltpu.async_copy(x_hbm.at[src_v[p]], rows_v[p], semg[p])

    def wait_gather(p):
        pltpu.make_async_copy(x_hbm.at[src_v[p]], rows_v[p], semg[p]).wait()

    def launch_scatter(p, q):
        pltpu.async_copy(rows_v[p], acc.at[dst_v[p]], sems[q], add=True)

    def wait_scatter(p, q):
        pltpu.make_async_copy(rows_v[p], acc.at[dst_v[p]], sems[q]).wait()

    def hist(p):
        for j in range(_WIN // 16):
            dvec = dst_v[p][pl.ds(j * 16, 16)]
            plsc.addupdate_scatter(cnt_v, [dvec], ones16)

    for w in (0, 1):
        extract(w, w)
        launch_gather(w)

    def step(v, p):
        wait_gather(p)
        hist(p)

        @pl.when(v >= 2)
        def _():
            wait_scatter((p + 2) % 4, p % 2)

        launch_scatter(p, p % 2)

        @pl.when(v + 2 < _NWIN)
        def _():
            extract(v + 2, (p + 2) % 4)
            launch_gather((p + 2) % 4)

    def quad(i, carry):
        v = 4 * i
        step(v, 0)
        step(v + 1, 1)
        step(v + 2, 2)
        step(v + 3, 3)
        return carry

    lax.fori_loop(0, _NWIN // 4, quad, 0)
    wait_scatter(2, 0)
    wait_scatter(3, 1)

    plsc.subcore_barrier()
    pltpu.sync_copy(acc.at[pl.ds(s * _RPT, _RPT)],
                    psum_hbm.at[c, pl.ds(s * _RPT, _RPT)])
    pltpu.sync_copy(cnt_v, cnt_hbm.at[wid])


def _tc_body(psum_ref, cnts_ref, x_ref, wlt_ref, wrt_ref, bl_ref, g_ref,
             b_ref, o_ref):
    ones = jnp.ones((_NW, 1), jnp.float32)
    cnt = lax.dot_general(cnts_ref[...], ones, (((0,), (0,)), ((), ())),
                          preferred_element_type=jnp.float32)
    p = (psum_ref[0].astype(jnp.float32)
         + psum_ref[1].astype(jnp.float32))
    agg = p / jnp.maximum(cnt, 1.0)
    h = (jnp.dot(agg, wlt_ref[...], preferred_element_type=jnp.float32)
         + jnp.dot(x_ref[...], wrt_ref[...], preferred_element_type=jnp.float32)
         + bl_ref[...])
    mean = jnp.mean(h, axis=1, keepdims=True)
    d = h - mean
    var = jnp.mean(d * d, axis=1, keepdims=True)
    hn = d * lax.rsqrt(var + 1e-5) * g_ref[...] + b_ref[...]
    o_ref[...] = 0.5 * hn * (1.0 + lax.erf(hn * 0.7071067811865476))


def _tc_epilogue(psum, cnts, x, wlt, wrt, bl, g, b):
    bn = 1024
    grid = ((N_NODES + bn - 1) // bn,)
    return pl.pallas_call(
        _tc_body,
        grid=grid,
        in_specs=[
            pl.BlockSpec((_NC, bn, D), lambda i: (0, i, 0)),
            pl.BlockSpec((_NW, bn), lambda i: (0, i)),
            pl.BlockSpec((bn, D), lambda i: (i, 0)),
            pl.BlockSpec((D, D), lambda i: (0, 0)),
            pl.BlockSpec((D, D), lambda i: (0, 0)),
            pl.BlockSpec((1, D), lambda i: (0, 0)),
            pl.BlockSpec((1, D), lambda i: (0, 0)),
            pl.BlockSpec((1, D), lambda i: (0, 0)),
        ],
        out_specs=pl.BlockSpec((bn, D), lambda i: (i, 0)),
        out_shape=jax.ShapeDtypeStruct((N_NODES, D), jnp.float32),
    )(psum, cnts, x, wlt, wrt, bl, g, b)


def kernel(x, edge_index, W_l, b_l, W_r, ln_gamma, ln_beta):
    ei = edge_index.astype(jnp.int32)
    packed = ei[0] | (ei[1] << _SHIFT)
    npad_e = _EPAD - N_EDGES
    pad_row = N_NODES + (jnp.arange(npad_e, dtype=jnp.int32) % 240)
    packed = jnp.concatenate([packed, pad_row | (pad_row << _SHIFT)])
    xb = jnp.zeros((_NPAD, D), jnp.bfloat16).at[:N_NODES].set(
        x.astype(jnp.bfloat16))
    zeros = jnp.zeros((_RPT, D), jnp.bfloat16)
    psum, cnts = _sc_segment_sum(xb, packed, zeros)
    return _tc_epilogue(psum, cnts, x, W_l.T, W_r.T,
                        b_l.reshape(1, D), ln_gamma.reshape(1, D),
                        ln_beta.reshape(1, D))


# speedup vs baseline: 1.4050x; 1.0006x over previous
import functools

import jax
import jax.numpy as jnp
from jax import lax
from jax.experimental import pallas as pl
from jax.experimental.pallas import tpu as pltpu
from jax.experimental.pallas import tpu_sc as plsc

N_NODES = 10000
N_EDGES = 320000
D = 128

_NC = 2
_NS = 16
_NW = _NC * _NS
_WIN = 128
_NWIN = 80
_EPT = _WIN * _NWIN
_EPAD = _NW * _EPT
_NPAD = 10240
_RPT = _NPAD // _NS
_SHIFT = 14


@functools.partial(
    pl.kernel,
    out_type=(
        jax.ShapeDtypeStruct((_NC, _NPAD, D), jnp.bfloat16),
        jax.ShapeDtypeStruct((_NW, _NPAD), jnp.float32),
    ),
    mesh=plsc.VectorSubcoreMesh(core_axis_name="c", subcore_axis_name="s"),
    compiler_params=pltpu.CompilerParams(needs_layout_passes=False,
                                         use_tc_tiling_on_sc=False),
    scratch_types=[
        pltpu.VMEM((_EPT,), jnp.int32),
    ] + [pltpu.VMEM((_WIN,), jnp.int32) for _ in range(4)]
      + [pltpu.VMEM((_WIN,), jnp.int32) for _ in range(4)]
      + [pltpu.VMEM((_WIN, D), jnp.bfloat16) for _ in range(4)]
      + [
        pltpu.VMEM((_NPAD,), jnp.float32),
        pltpu.VMEM_SHARED((_NPAD, D), jnp.bfloat16),
    ] + [pltpu.SemaphoreType.DMA for _ in range(4)]
      + [pltpu.SemaphoreType.DMA for _ in range(2)],
)
def _sc_segment_sum(x_hbm, pk_hbm, zeros_hbm, psum_hbm, cnt_hbm,
                    pk_t, s0, s1, s2, s3, d0, d1, d2, d3, r0, r1, r2, r3,
                    cnt_v, acc, g0, g1, g2, g3, t0, t1):
    c = lax.axis_index("c")
    s = lax.axis_index("s")
    wid = c * _NS + s
    base0 = wid * _EPT
    src_v = (s0, s1, s2, s3)
    dst_v = (d0, d1, d2, d3)
    rows_v = (r0, r1, r2, r3)
    semg = (g0, g1, g2, g3)
    sems = (t0, t1)

    nreal = N_EDGES // _NW
    pltpu.async_copy(pk_hbm.at[pl.ds(wid * nreal, nreal)],
                     pk_t.at[pl.ds(0, nreal)], g0)
    pltpu.sync_copy(zeros_hbm, acc.at[pl.ds(s * _RPT, _RPT)])

    iota16 = lax.iota(jnp.int32, 16)
    npad = _EPT - nreal
    for k in range(npad // 16):
        srcv = iota16 + (wid * npad + k * 16)
        dstv = iota16 + (N_NODES + k * 16)
        pk_t[pl.ds(nreal + k * 16, 16)] = lax.bitwise_or(
            srcv, lax.shift_left(dstv, _SHIFT))

    zeros16 = jnp.zeros((16,), jnp.float32)

    def zbody(i, carry):
        cnt_v[pl.ds(i * 16, 16)] = zeros16
        return carry

    lax.fori_loop(0, _NPAD // 16, zbody, 0)
    pltpu.make_async_copy(pk_hbm.at[pl.ds(wid * nreal, nreal)],
                          pk_t.at[pl.ds(0, nreal)], g0).wait()
    plsc.subcore_barrier()

    ones16 = jnp.ones((16,), jnp.float32)
    mask14 = jnp.full((16,), (1 << _SHIFT) - 1, jnp.int32)

    def extract(w, p):
        for j in range(_WIN // 16):
            pk = pk_t[pl.ds(w * _WIN + j * 16, 16)]
            src_v[p][pl.ds(j * 16, 16)] = lax.bitwise_and(pk, mask14)
            dst_v[p][pl.ds(j * 16, 16)] = lax.shift_right_logical(pk, _SHIFT)

    def launch_gather(p):
        pltpu.async_copy(x_hbm.at[src_v[p]], rows_v[p], semg[p])

    def wait_gather(p):
        pltpu.make_async_copy(x_hbm.at[src_v[p]], rows_v[p], semg[p]).wait()

    def launch_scatter(p, q):
        pltpu.async_copy(rows_v[p], acc.at[dst_v[p]], sems[q], add=True)

    def wait_scatter(p, q):
        pltpu.make_async_copy(rows_v[p], acc.at[dst_v[p]], sems[q]).wait()

    def hist(p):
        for j in range(_WIN // 16):
            dvec = dst_v[p][pl.ds(j * 16, 16)]
            plsc.addupdate_scatter(cnt_v, [dvec], ones16)

    for w in (0, 1):
        extract(w, w)
        launch_gather(w)

    def step(v, p):
        wait_gather(p)
        hist(p)

        @pl.when(v >= 2)
        def _():
            wait_scatter((p + 2) % 4, p % 2)

        launch_scatter(p, p % 2)

        @pl.when(v + 2 < _NWIN)
        def _():
            extract(v + 2, (p + 2) % 4)
            launch_gather((p + 2) % 4)

    def quad(i, carry):
        v = 4 * i
        step(v, 0)
        step(v + 1, 1)
        step(v + 2, 2)
        step(v + 3, 3)
        return carry

    lax.fori_loop(0, _NWIN // 4, quad, 0)
    wait_scatter(2, 0)
    wait_scatter(3, 1)

    plsc.subcore_barrier()
    pltpu.sync_copy(acc.at[pl.ds(s * _RPT, _RPT)],
                    psum_hbm.at[c, pl.ds(s * _RPT, _RPT)])
    pltpu.sync_copy(cnt_v, cnt_hbm.at[wid])


def _tc_body(psum_ref, cnts_ref, x_ref, wlt_ref, wrt_ref, bl_ref, g_ref,
             b_ref, o_ref):
    ones = jnp.ones((_NW, 1), jnp.float32)
    cnt = lax.dot_general(cnts_ref[...], ones, (((0,), (0,)), ((), ())),
                          preferred_element_type=jnp.float32)
    p = (psum_ref[0].astype(jnp.float32)
         + psum_ref[1].astype(jnp.float32))
    agg = p / jnp.maximum(cnt, 1.0)
    h = (jnp.dot(agg, wlt_ref[...], preferred_element_type=jnp.float32)
         + jnp.dot(x_ref[...], wrt_ref[...], preferred_element_type=jnp.float32)
         + bl_ref[...])
    mean = jnp.mean(h, axis=1, keepdims=True)
    d = h - mean
    var = jnp.mean(d * d, axis=1, keepdims=True)
    hn = d * lax.rsqrt(var + 1e-5) * g_ref[...] + b_ref[...]
    o_ref[...] = 0.5 * hn * (1.0 + lax.erf(hn * 0.7071067811865476))


def _tc_epilogue(psum, cnts, x, wlt, wrt, bl, g, b):
    bn = 1024
    grid = ((N_NODES + bn - 1) // bn,)
    return pl.pallas_call(
        _tc_body,
        grid=grid,
        in_specs=[
            pl.BlockSpec((_NC, bn, D), lambda i: (0, i, 0)),
            pl.BlockSpec((_NW, bn), lambda i: (0, i)),
            pl.BlockSpec((bn, D), lambda i: (i, 0)),
            pl.BlockSpec((D, D), lambda i: (0, 0)),
            pl.BlockSpec((D, D), lambda i: (0, 0)),
            pl.BlockSpec((1, D), lambda i: (0, 0)),
            pl.BlockSpec((1, D), lambda i: (0, 0)),
            pl.BlockSpec((1, D), lambda i: (0, 0)),
        ],
        out_specs=pl.BlockSpec((bn, D), lambda i: (i, 0)),
        out_shape=jax.ShapeDtypeStruct((N_NODES, D), jnp.float32),
    )(psum, cnts, x, wlt, wrt, bl, g, b)


def kernel(x, edge_index, W_l, b_l, W_r, ln_gamma, ln_beta):
    ei = edge_index.astype(jnp.int32)
    packed = ei[0] | (ei[1] << _SHIFT)
    zeros = jnp.zeros((_RPT, D), jnp.bfloat16)
    psum, cnts = _sc_segment_sum(x.astype(jnp.bfloat16), packed, zeros)
    return _tc_epilogue(psum, cnts, x, W_l.T, W_r.T,
                        b_l.reshape(1, D), ln_gamma.reshape(1, D),
                        ln_beta.reshape(1, D))
